# Initial kernel scaffold; baseline (speedup 1.0000x reference)
import jax
import jax.numpy as jnp
from jax.experimental import pallas as pl


def kernel(timestamps, labels, seq_lens, predictions):
    raise NotImplementedError("write your pallas kernel here")


# speedup vs baseline: 1.4007x; 1.4007x over previous
import jax
import jax.numpy as jnp
from jax.experimental import pallas as pl
from jax.experimental.pallas import tpu as pltpu

K = 8
NUM_CLASSES = 64
INPUT_DIM = 1 + NUM_CLASSES
B, L = 128, 512
LMAX = L - K
D = K * INPUT_DIM


def _tc_body(seq_ref, pred_ref, dmat_ref, ltgt_ref, ssel_ref, rrep_ref,
             dsel_ref, out_ref):
    b = pl.program_id(0)

    @pl.when(b == 0)
    def _init():
        out_ref[0] = 0.0
        out_ref[1] = 0.0
        out_ref[2] = 0.0

    length = jnp.maximum(seq_ref[b] - K, 0)
    x = pred_ref[0]
    dmat = dmat_ref[0]
    ltgt = ltgt_ref[0]

    ones_m = (jax.lax.broadcasted_iota(jnp.int32, (1, L), 1)
              < length).astype(jnp.float32)

    e = jnp.exp(x)
    sums = jax.lax.dot_general(
        ssel_ref[...], e, (((1,), (1,)), ((), ())),
        preferred_element_type=jnp.float32)
    lse = jnp.log(sums)
    lmask = (jax.lax.broadcasted_iota(jnp.int32, (K, L), 1)
             < length)
    acc_lse = jnp.sum(jnp.where(lmask, lse, 0.0))

    ltl = jax.lax.dot_general(
        ltgt, rrep_ref[...], (((0,), (0,)), ((), ())),
        preferred_element_type=jnp.float32)
    lane = jax.lax.broadcasted_iota(jnp.int32, (L, D), 1
                                    ).astype(jnp.float32)
    sel = jnp.where(ltl == lane, x, 0.0)
    ones_all = jnp.zeros((1, L), jnp.float32) + 1.0
    p1 = jax.lax.dot_general(
        ones_all, sel, (((1,), (0,)), ((), ())),
        preferred_element_type=jnp.float32)
    acc_pick = jnp.sum(p1)

    deltal = jax.lax.dot_general(
        dmat, dsel_ref[...], (((0,), (0,)), ((), ())),
        preferred_element_type=jnp.float32)
    sq = (x - deltal) ** 2
    t1 = jax.lax.dot_general(
        ones_m, sq, (((1,), (0,)), ((), ())),
        preferred_element_type=jnp.float32)
    dany = jnp.sum(dsel_ref[...], axis=0, keepdims=True)
    acc_ts = jnp.sum(t1 * dany)

    out_ref[0] += acc_ts
    out_ref[1] += acc_lse - acc_pick
    out_ref[2] += length.astype(jnp.float32)


def _build_consts():
    c = jnp.arange(D, dtype=jnp.int32)
    seg = c // INPUT_DIM
    off = c % INPUT_DIM
    t = jnp.arange(K, dtype=jnp.int32)
    ssel = ((seg[None, :] == t[:, None]) & (off[None, :] > 0)
            ).astype(jnp.float32)
    rrep = (seg[None, :] == t[:, None]).astype(jnp.float32)
    dsel = ((seg[None, :] == t[:, None]) & (off[None, :] == 0)
            ).astype(jnp.float32)
    return ssel, rrep, dsel


def _build_aux(timestamps, labels, seq_lens):
    lengths = jnp.clip(seq_lens - K, 0, None)
    i = jnp.arange(L, dtype=jnp.int32)
    valid = (i[None, None, :] < lengths[:, None, None]) & \
            (i[None, None, :] < LMAX)
    t = jnp.arange(K, dtype=jnp.int32)
    idx = jnp.minimum(i[None, :] + t[:, None], L - 1)
    idx1 = jnp.minimum(i[None, :] + t[:, None] + 1, L - 1)
    ts_a = timestamps[:, idx]
    ts_b = timestamps[:, idx1]
    dmat = jnp.where(valid, ts_b - ts_a, 0.0)
    lt = (t[:, None] * INPUT_DIM + 1)[None] + labels[:, idx1]
    ltgt = jnp.where(valid, lt.astype(jnp.float32), -1000.0)
    return dmat, ltgt


def kernel(timestamps, labels, seq_lens, predictions):
    ssel, rrep, dsel = _build_consts()
    dmat, ltgt = _build_aux(timestamps, labels, seq_lens)
    sums = pl.pallas_call(
        _tc_body,
        grid=(B,),
        in_specs=[
            pl.BlockSpec(memory_space=pltpu.SMEM),
            pl.BlockSpec((1, L, D), lambda b: (b, 0, 0)),
            pl.BlockSpec((1, K, L), lambda b: (b, 0, 0)),
            pl.BlockSpec((1, K, L), lambda b: (b, 0, 0)),
            pl.BlockSpec((K, D), lambda b: (0, 0)),
            pl.BlockSpec((K, D), lambda b: (0, 0)),
            pl.BlockSpec((K, D), lambda b: (0, 0)),
        ],
        out_specs=pl.BlockSpec(memory_space=pltpu.SMEM),
        out_shape=jax.ShapeDtypeStruct((3,), jnp.float32),
    )(seq_lens.astype(jnp.int32), predictions, dmat, ltgt, ssel, rrep, dsel)
    denom = sums[2] * jnp.float32(K)
    return jnp.stack([sums[0], sums[1]]) / denom


# speedup vs baseline: 1.5523x; 1.1082x over previous
import functools

import jax
import jax.numpy as jnp
from jax import lax
from jax.experimental import pallas as pl
from jax.experimental.pallas import tpu as pltpu
from jax.experimental.pallas import tpu_sc as plsc

K = 8
NUM_CLASSES = 64
INPUT_DIM = 1 + NUM_CLASSES
B, L = 128, 512
LMAX = L - K
D = K * INPUT_DIM


def _tc_body(seq_ref, pred_ref, dmat_ref, ltgt_ref, ssel_ref, rrep_ref,
             dsel_ref, out_ref):
    b = pl.program_id(0)

    @pl.when(b == 0)
    def _init():
        out_ref[0] = 0.0
        out_ref[1] = 0.0
        out_ref[2] = 0.0

    length = jnp.maximum(seq_ref[b] - K, 0)
    x = pred_ref[0]
    dmat = dmat_ref[0]
    ltgt = ltgt_ref[0]

    ones_m = (jax.lax.broadcasted_iota(jnp.int32, (1, L), 1)
              < length).astype(jnp.float32)

    e = jnp.exp(x)
    sums = jax.lax.dot_general(
        ssel_ref[...], e, (((1,), (1,)), ((), ())),
        preferred_element_type=jnp.float32)
    lse = jnp.log(sums)
    lmask = (jax.lax.broadcasted_iota(jnp.int32, (K, L), 1)
             < length)
    acc_lse = jnp.sum(jnp.where(lmask, lse, 0.0))

    ltl = jax.lax.dot_general(
        ltgt, rrep_ref[...], (((0,), (0,)), ((), ())),
        preferred_element_type=jnp.float32)
    lane = jax.lax.broadcasted_iota(jnp.int32, (L, D), 1
                                    ).astype(jnp.float32)
    sel = jnp.where(ltl == lane, x, 0.0)
    ones_all = jnp.zeros((1, L), jnp.float32) + 1.0
    p1 = jax.lax.dot_general(
        ones_all, sel, (((1,), (0,)), ((), ())),
        preferred_element_type=jnp.float32)
    acc_pick = jnp.sum(p1)

    deltal = jax.lax.dot_general(
        dmat, dsel_ref[...], (((0,), (0,)), ((), ())),
        preferred_element_type=jnp.float32)
    sq = (x - deltal) ** 2
    t1 = jax.lax.dot_general(
        ones_m, sq, (((1,), (0,)), ((), ())),
        preferred_element_type=jnp.float32)
    dany = jnp.sum(dsel_ref[...], axis=0, keepdims=True)
    acc_ts = jnp.sum(t1 * dany)

    out_ref[0] += acc_ts
    out_ref[1] += acc_lse - acc_pick
    out_ref[2] += length.astype(jnp.float32)


def _build_consts():
    c = jnp.arange(D, dtype=jnp.int32)
    seg = c // INPUT_DIM
    off = c % INPUT_DIM
    t = jnp.arange(K, dtype=jnp.int32)
    ssel = ((seg[None, :] == t[:, None]) & (off[None, :] > 0)
            ).astype(jnp.float32)
    rrep = (seg[None, :] == t[:, None]).astype(jnp.float32)
    dsel = ((seg[None, :] == t[:, None]) & (off[None, :] == 0)
            ).astype(jnp.float32)
    return ssel, rrep, dsel


_VB = 16
_TSBUF = L + K


def _sc_body(ts_hbm, lbl_hbm, seq_hbm, dmat_hbm, ltgt_hbm,
             ts_v, lbl_v, seq_v, dm_v, lt_v):
    nc = 2
    wid = lax.axis_index("s") * nc + lax.axis_index("c")
    rpw = B // 32
    iota = lax.iota(jnp.int32, _VB)
    zf = jnp.zeros((_VB,), jnp.float32)
    zi = jnp.zeros((_VB,), jnp.int32)
    pltpu.sync_copy(seq_hbm, seq_v.at[pl.ds(0, B)])
    for j in range(rpw):
        row = wid * rpw + j
        ts_v[pl.ds(L - _VB + K, _VB)] = zf
        lbl_v[pl.ds(L - _VB + K, _VB)] = zi
        pltpu.sync_copy(ts_hbm.at[row], ts_v.at[pl.ds(0, L)])
        pltpu.sync_copy(lbl_hbm.at[row], lbl_v.at[pl.ds(0, L)])
        length = seq_v[pl.ds(row, _VB)][0] - K

        def body(k, carry):
            i0 = k * _VB
            valid = (i0 + iota) < length
            for t in range(K):
                a = ts_v[pl.ds(i0 + t, _VB)]
                b2 = ts_v[pl.ds(i0 + t + 1, _VB)]
                lbl = lbl_v[pl.ds(i0 + t + 1, _VB)]
                dm_v[t, pl.ds(i0, _VB)] = jnp.where(valid, b2 - a, 0.0)
                lt = (lbl + (t * INPUT_DIM + 1)).astype(jnp.float32)
                lt_v[t, pl.ds(i0, _VB)] = jnp.where(valid, lt, -1000.0)
            return carry

        lax.fori_loop(0, L // _VB, body, 0)
        pltpu.sync_copy(dm_v, dmat_hbm.at[row])
        pltpu.sync_copy(lt_v, ltgt_hbm.at[row])


@functools.partial(
    pl.kernel,
    mesh=plsc.VectorSubcoreMesh(core_axis_name="c", subcore_axis_name="s"),
    out_type=[
        jax.ShapeDtypeStruct((B, K, L), jnp.float32),
        jax.ShapeDtypeStruct((B, K, L), jnp.float32),
    ],
    scratch_types=[
        pltpu.VMEM((_TSBUF,), jnp.float32),
        pltpu.VMEM((_TSBUF,), jnp.int32),
        pltpu.VMEM((B + _VB,), jnp.int32),
        pltpu.VMEM((K, L), jnp.float32),
        pltpu.VMEM((K, L), jnp.float32),
    ],
)
def _build_aux_sc(ts_hbm, lbl_hbm, seq_hbm, dmat_hbm, ltgt_hbm,
                  ts_v, lbl_v, seq_v, dm_v, lt_v):
    _sc_body(ts_hbm, lbl_hbm, seq_hbm, dmat_hbm, ltgt_hbm,
             ts_v, lbl_v, seq_v, dm_v, lt_v)


def kernel(timestamps, labels, seq_lens, predictions):
    ssel, rrep, dsel = _build_consts()
    dmat, ltgt = _build_aux_sc(timestamps, labels.astype(jnp.int32),
                               seq_lens.astype(jnp.int32))
    sums = pl.pallas_call(
        _tc_body,
        grid=(B,),
        in_specs=[
            pl.BlockSpec(memory_space=pltpu.SMEM),
            pl.BlockSpec((1, L, D), lambda b: (b, 0, 0)),
            pl.BlockSpec((1, K, L), lambda b: (b, 0, 0)),
            pl.BlockSpec((1, K, L), lambda b: (b, 0, 0)),
            pl.BlockSpec((K, D), lambda b: (0, 0)),
            pl.BlockSpec((K, D), lambda b: (0, 0)),
            pl.BlockSpec((K, D), lambda b: (0, 0)),
        ],
        out_specs=pl.BlockSpec(memory_space=pltpu.SMEM),
        out_shape=jax.ShapeDtypeStruct((3,), jnp.float32),
    )(seq_lens.astype(jnp.int32), predictions, dmat, ltgt, ssel, rrep, dsel)
    denom = sums[2] * jnp.float32(K)
    return jnp.stack([sums[0], sums[1]]) / denom


# speedup vs baseline: 1.7105x; 1.1019x over previous
import functools

import jax
import jax.numpy as jnp
from jax import lax
from jax.experimental import pallas as pl
from jax.experimental.pallas import tpu as pltpu
from jax.experimental.pallas import tpu_sc as plsc

K = 8
NUM_CLASSES = 64
INPUT_DIM = 1 + NUM_CLASSES
B, L = 128, 512
LMAX = L - K
D = K * INPUT_DIM


ROWS_PER_STEP = 2


def _tc_body(seq_ref, pred_ref, dmat_ref, ltgt_ref, ssel_ref, rrep_ref,
             dsel_ref, out_ref):
    g = pl.program_id(0)

    @pl.when(g == 0)
    def _init():
        out_ref[0] = 0.0
        out_ref[1] = 0.0
        out_ref[2] = 0.0

    for r in range(ROWS_PER_STEP):
        _tc_row(seq_ref, pred_ref, dmat_ref, ltgt_ref, ssel_ref, rrep_ref,
                dsel_ref, out_ref, g * ROWS_PER_STEP + r, r)


def _tc_row(seq_ref, pred_ref, dmat_ref, ltgt_ref, ssel_ref, rrep_ref,
            dsel_ref, out_ref, b, r):
    length = jnp.maximum(seq_ref[b] - K, 0)
    x = pred_ref[r]
    dmat = dmat_ref[r]
    ltgt = ltgt_ref[r]

    ones_m = (jax.lax.broadcasted_iota(jnp.int32, (1, L), 1)
              < length).astype(jnp.float32)

    e = jnp.exp(x)
    sums = jax.lax.dot_general(
        ssel_ref[...], e, (((1,), (1,)), ((), ())),
        preferred_element_type=jnp.float32)
    lse = jnp.log(sums)
    lmask = (jax.lax.broadcasted_iota(jnp.int32, (K, L), 1)
             < length)
    acc_lse = jnp.sum(jnp.where(lmask, lse, 0.0))

    ltl = jax.lax.dot_general(
        ltgt, rrep_ref[...], (((0,), (0,)), ((), ())),
        preferred_element_type=jnp.float32)
    lane = jax.lax.broadcasted_iota(jnp.int32, (L, D), 1
                                    ).astype(jnp.float32)
    sel = jnp.where(ltl == lane, x, 0.0)
    ones_all = jnp.zeros((1, L), jnp.float32) + 1.0
    p1 = jax.lax.dot_general(
        ones_all, sel, (((1,), (0,)), ((), ())),
        preferred_element_type=jnp.float32)
    acc_pick = jnp.sum(p1)

    deltal = jax.lax.dot_general(
        dmat, dsel_ref[...], (((0,), (0,)), ((), ())),
        preferred_element_type=jnp.float32)
    sq = (x - deltal) ** 2
    t1 = jax.lax.dot_general(
        ones_m, sq, (((1,), (0,)), ((), ())),
        preferred_element_type=jnp.float32)
    dany = jnp.sum(dsel_ref[...], axis=0, keepdims=True)
    acc_ts = jnp.sum(t1 * dany)

    out_ref[0] += acc_ts
    out_ref[1] += acc_lse - acc_pick
    out_ref[2] += length.astype(jnp.float32)


def _build_consts():
    c = jnp.arange(D, dtype=jnp.int32)
    seg = c // INPUT_DIM
    off = c % INPUT_DIM
    t = jnp.arange(K, dtype=jnp.int32)
    ssel = ((seg[None, :] == t[:, None]) & (off[None, :] > 0)
            ).astype(jnp.float32)
    rrep = (seg[None, :] == t[:, None]).astype(jnp.float32)
    dsel = ((seg[None, :] == t[:, None]) & (off[None, :] == 0)
            ).astype(jnp.float32)
    return ssel, rrep, dsel


_VB = 16
_TSBUF = L + K


def _sc_body(ts_hbm, lbl_hbm, seq_hbm, dmat_hbm, ltgt_hbm,
             ts_v, lbl_v, seq_v, dm_v, lt_v):
    nc = 2
    wid = lax.axis_index("s") * nc + lax.axis_index("c")
    rpw = B // 32
    iota = lax.iota(jnp.int32, _VB)
    zf = jnp.zeros((_VB,), jnp.float32)
    zi = jnp.zeros((_VB,), jnp.int32)
    pltpu.sync_copy(seq_hbm, seq_v.at[pl.ds(0, B)])
    for j in range(rpw):
        row = wid * rpw + j
        ts_v[pl.ds(L - _VB + K, _VB)] = zf
        lbl_v[pl.ds(L - _VB + K, _VB)] = zi
        pltpu.sync_copy(ts_hbm.at[row], ts_v.at[pl.ds(0, L)])
        pltpu.sync_copy(lbl_hbm.at[row], lbl_v.at[pl.ds(0, L)])
        length = seq_v[pl.ds(row, _VB)][0] - K

        def body(k, carry):
            i0 = k * _VB
            valid = (i0 + iota) < length
            for t in range(K):
                a = ts_v[pl.ds(i0 + t, _VB)]
                b2 = ts_v[pl.ds(i0 + t + 1, _VB)]
                lbl = lbl_v[pl.ds(i0 + t + 1, _VB)]
                dm_v[t, pl.ds(i0, _VB)] = jnp.where(valid, b2 - a, 0.0)
                lt = (lbl + (t * INPUT_DIM + 1)).astype(jnp.float32)
                lt_v[t, pl.ds(i0, _VB)] = jnp.where(valid, lt, -1000.0)
            return carry

        lax.fori_loop(0, L // _VB, body, 0)
        pltpu.sync_copy(dm_v, dmat_hbm.at[row])
        pltpu.sync_copy(lt_v, ltgt_hbm.at[row])


@functools.partial(
    pl.kernel,
    mesh=plsc.VectorSubcoreMesh(core_axis_name="c", subcore_axis_name="s"),
    out_type=[
        jax.ShapeDtypeStruct((B, K, L), jnp.float32),
        jax.ShapeDtypeStruct((B, K, L), jnp.float32),
    ],
    scratch_types=[
        pltpu.VMEM((_TSBUF,), jnp.float32),
        pltpu.VMEM((_TSBUF,), jnp.int32),
        pltpu.VMEM((B + _VB,), jnp.int32),
        pltpu.VMEM((K, L), jnp.float32),
        pltpu.VMEM((K, L), jnp.float32),
    ],
)
def _build_aux_sc(ts_hbm, lbl_hbm, seq_hbm, dmat_hbm, ltgt_hbm,
                  ts_v, lbl_v, seq_v, dm_v, lt_v):
    _sc_body(ts_hbm, lbl_hbm, seq_hbm, dmat_hbm, ltgt_hbm,
             ts_v, lbl_v, seq_v, dm_v, lt_v)


def kernel(timestamps, labels, seq_lens, predictions):
    ssel, rrep, dsel = _build_consts()
    dmat, ltgt = _build_aux_sc(timestamps, labels.astype(jnp.int32),
                               seq_lens.astype(jnp.int32))
    sums = pl.pallas_call(
        _tc_body,
        grid=(B // ROWS_PER_STEP,),
        in_specs=[
            pl.BlockSpec(memory_space=pltpu.SMEM),
            pl.BlockSpec((ROWS_PER_STEP, L, D), lambda b: (b, 0, 0)),
            pl.BlockSpec((ROWS_PER_STEP, K, L), lambda b: (b, 0, 0)),
            pl.BlockSpec((ROWS_PER_STEP, K, L), lambda b: (b, 0, 0)),
            pl.BlockSpec((K, D), lambda b: (0, 0)),
            pl.BlockSpec((K, D), lambda b: (0, 0)),
            pl.BlockSpec((K, D), lambda b: (0, 0)),
        ],
        out_specs=pl.BlockSpec(memory_space=pltpu.SMEM),
        out_shape=jax.ShapeDtypeStruct((3,), jnp.float32),
    )(seq_lens.astype(jnp.int32), predictions, dmat, ltgt, ssel, rrep, dsel)
    denom = sums[2] * jnp.float32(K)
    return jnp.stack([sums[0], sums[1]]) / denom


# speedup vs baseline: 1.8046x; 1.0551x over previous
import functools

import jax
import jax.numpy as jnp
from jax import lax
from jax.experimental import pallas as pl
from jax.experimental.pallas import tpu as pltpu
from jax.experimental.pallas import tpu_sc as plsc

K = 8
NUM_CLASSES = 64
INPUT_DIM = 1 + NUM_CLASSES
B, L = 128, 512
LMAX = L - K
D = K * INPUT_DIM


ROWS_PER_STEP = 8


def _tc_body(seq_ref, pred_ref, dmat_ref, ltgt_ref, ssel_ref, rrep_ref,
             dsel_ref, out_ref):
    g = pl.program_id(0)

    @pl.when(g == 0)
    def _init():
        out_ref[0] = 0.0
        out_ref[1] = 0.0
        out_ref[2] = 0.0

    for r in range(ROWS_PER_STEP):
        _tc_row(seq_ref, pred_ref, dmat_ref, ltgt_ref, ssel_ref, rrep_ref,
                dsel_ref, out_ref, g * ROWS_PER_STEP + r, r)


def _tc_row(seq_ref, pred_ref, dmat_ref, ltgt_ref, ssel_ref, rrep_ref,
            dsel_ref, out_ref, b, r):
    length = jnp.maximum(seq_ref[b] - K, 0)
    x = pred_ref[r]
    dmat = dmat_ref[r]
    ltgt = ltgt_ref[r]

    ones_m = (jax.lax.broadcasted_iota(jnp.int32, (1, L), 1)
              < length).astype(jnp.float32)

    e = jnp.exp(x)
    sums = jax.lax.dot_general(
        ssel_ref[...], e, (((1,), (1,)), ((), ())),
        preferred_element_type=jnp.float32)
    lse = jnp.log(sums)
    lmask = (jax.lax.broadcasted_iota(jnp.int32, (K, L), 1)
             < length)
    acc_lse = jnp.sum(jnp.where(lmask, lse, 0.0))

    ltl = jax.lax.dot_general(
        ltgt, rrep_ref[...], (((0,), (0,)), ((), ())),
        preferred_element_type=jnp.float32)
    lane = jax.lax.broadcasted_iota(jnp.int32, (L, D), 1
                                    ).astype(jnp.float32)
    sel = jnp.where(ltl == lane, x, 0.0)
    ones_all = jnp.zeros((1, L), jnp.float32) + 1.0
    p1 = jax.lax.dot_general(
        ones_all, sel, (((1,), (0,)), ((), ())),
        preferred_element_type=jnp.float32)
    acc_pick = jnp.sum(p1)

    deltal = jax.lax.dot_general(
        dmat, dsel_ref[...], (((0,), (0,)), ((), ())),
        preferred_element_type=jnp.float32)
    sq = (x - deltal) ** 2
    t1 = jax.lax.dot_general(
        ones_m, sq, (((1,), (0,)), ((), ())),
        preferred_element_type=jnp.float32)
    dany = jnp.sum(dsel_ref[...], axis=0, keepdims=True)
    acc_ts = jnp.sum(t1 * dany)

    out_ref[0] += acc_ts
    out_ref[1] += acc_lse - acc_pick
    out_ref[2] += length.astype(jnp.float32)


def _build_consts():
    c = jnp.arange(D, dtype=jnp.int32)
    seg = c // INPUT_DIM
    off = c % INPUT_DIM
    t = jnp.arange(K, dtype=jnp.int32)
    ssel = ((seg[None, :] == t[:, None]) & (off[None, :] > 0)
            ).astype(jnp.float32)
    rrep = (seg[None, :] == t[:, None]).astype(jnp.float32)
    dsel = ((seg[None, :] == t[:, None]) & (off[None, :] == 0)
            ).astype(jnp.float32)
    return ssel, rrep, dsel


_VB = 16
_TSBUF = L + K


def _sc_body(ts_hbm, lbl_hbm, seq_hbm, dmat_hbm, ltgt_hbm,
             ts_v, lbl_v, seq_v, dm_v, lt_v):
    nc = 2
    wid = lax.axis_index("s") * nc + lax.axis_index("c")
    rpw = B // 32
    iota = lax.iota(jnp.int32, _VB)
    zf = jnp.zeros((_VB,), jnp.float32)
    zi = jnp.zeros((_VB,), jnp.int32)
    pltpu.sync_copy(seq_hbm, seq_v.at[pl.ds(0, B)])
    for j in range(rpw):
        row = wid * rpw + j
        ts_v[pl.ds(L - _VB + K, _VB)] = zf
        lbl_v[pl.ds(L - _VB + K, _VB)] = zi
        pltpu.sync_copy(ts_hbm.at[row], ts_v.at[pl.ds(0, L)])
        pltpu.sync_copy(lbl_hbm.at[row], lbl_v.at[pl.ds(0, L)])
        length = seq_v[pl.ds(row, _VB)][0] - K

        def body(k, carry):
            i0 = k * _VB
            valid = (i0 + iota) < length
            for t in range(K):
                a = ts_v[pl.ds(i0 + t, _VB)]
                b2 = ts_v[pl.ds(i0 + t + 1, _VB)]
                lbl = lbl_v[pl.ds(i0 + t + 1, _VB)]
                dm_v[t, pl.ds(i0, _VB)] = jnp.where(valid, b2 - a, 0.0)
                lt = (lbl + (t * INPUT_DIM + 1)).astype(jnp.float32)
                lt_v[t, pl.ds(i0, _VB)] = jnp.where(valid, lt, -1000.0)
            return carry

        lax.fori_loop(0, L // _VB, body, 0)
        pltpu.sync_copy(dm_v, dmat_hbm.at[row])
        pltpu.sync_copy(lt_v, ltgt_hbm.at[row])


@functools.partial(
    pl.kernel,
    mesh=plsc.VectorSubcoreMesh(core_axis_name="c", subcore_axis_name="s"),
    out_type=[
        jax.ShapeDtypeStruct((B, K, L), jnp.float32),
        jax.ShapeDtypeStruct((B, K, L), jnp.float32),
    ],
    scratch_types=[
        pltpu.VMEM((_TSBUF,), jnp.float32),
        pltpu.VMEM((_TSBUF,), jnp.int32),
        pltpu.VMEM((B + _VB,), jnp.int32),
        pltpu.VMEM((K, L), jnp.float32),
        pltpu.VMEM((K, L), jnp.float32),
    ],
)
def _build_aux_sc(ts_hbm, lbl_hbm, seq_hbm, dmat_hbm, ltgt_hbm,
                  ts_v, lbl_v, seq_v, dm_v, lt_v):
    _sc_body(ts_hbm, lbl_hbm, seq_hbm, dmat_hbm, ltgt_hbm,
             ts_v, lbl_v, seq_v, dm_v, lt_v)


def kernel(timestamps, labels, seq_lens, predictions):
    ssel, rrep, dsel = _build_consts()
    dmat, ltgt = _build_aux_sc(timestamps, labels.astype(jnp.int32),
                               seq_lens.astype(jnp.int32))
    sums = pl.pallas_call(
        _tc_body,
        grid=(B // ROWS_PER_STEP,),
        in_specs=[
            pl.BlockSpec(memory_space=pltpu.SMEM),
            pl.BlockSpec((ROWS_PER_STEP, L, D), lambda b: (b, 0, 0)),
            pl.BlockSpec((ROWS_PER_STEP, K, L), lambda b: (b, 0, 0)),
            pl.BlockSpec((ROWS_PER_STEP, K, L), lambda b: (b, 0, 0)),
            pl.BlockSpec((K, D), lambda b: (0, 0)),
            pl.BlockSpec((K, D), lambda b: (0, 0)),
            pl.BlockSpec((K, D), lambda b: (0, 0)),
        ],
        out_specs=pl.BlockSpec(memory_space=pltpu.SMEM),
        out_shape=jax.ShapeDtypeStruct((3,), jnp.float32),
    )(seq_lens.astype(jnp.int32), predictions, dmat, ltgt, ssel, rrep, dsel)
    denom = sums[2] * jnp.float32(K)
    return jnp.stack([sums[0], sums[1]]) / denom


# speedup vs baseline: 1.8274x; 1.0126x over previous
import functools

import jax
import jax.numpy as jnp
from jax import lax
from jax.experimental import pallas as pl
from jax.experimental.pallas import tpu as pltpu
from jax.experimental.pallas import tpu_sc as plsc

K = 8
NUM_CLASSES = 64
INPUT_DIM = 1 + NUM_CLASSES
B, L = 128, 512
LMAX = L - K
D = K * INPUT_DIM


ROWS_PER_STEP = 8


def _tc_body(seq_ref, pred_ref, dmat_ref, ltgt_ref, ssel_ref, rrep_ref,
             dsel_ref, out_ref):
    g = pl.program_id(0)

    @pl.when(g == 0)
    def _init():
        out_ref[...] = jnp.zeros((4, D), jnp.float32)

    for r in range(ROWS_PER_STEP):
        _tc_row(seq_ref, pred_ref, dmat_ref, ltgt_ref, ssel_ref, rrep_ref,
                dsel_ref, out_ref, g * ROWS_PER_STEP + r, r)


def _tc_row(seq_ref, pred_ref, dmat_ref, ltgt_ref, ssel_ref, rrep_ref,
            dsel_ref, out_ref, b, r):
    length = jnp.maximum(seq_ref[b] - K, 0)
    x = pred_ref[r]
    dmat = dmat_ref[r]
    ltgt = ltgt_ref[r]

    ones_m = (jax.lax.broadcasted_iota(jnp.int32, (1, L), 1)
              < length).astype(jnp.float32)

    e = jnp.exp(x)
    sums = jax.lax.dot_general(
        ssel_ref[...], e, (((1,), (1,)), ((), ())),
        preferred_element_type=jnp.float32)
    lse = jnp.log(sums)
    lmask = (jax.lax.broadcasted_iota(jnp.int32, (K, L), 1)
             < length)
    lse_row = jnp.sum(jnp.where(lmask, lse, 0.0), axis=0,
                      keepdims=True)

    ltl = jax.lax.dot_general(
        ltgt, rrep_ref[...], (((0,), (0,)), ((), ())),
        preferred_element_type=jnp.float32)
    lane = jax.lax.broadcasted_iota(jnp.int32, (L, D), 1
                                    ).astype(jnp.float32)
    sel = jnp.where(ltl == lane, x, 0.0)
    ones_all = jnp.zeros((1, L), jnp.float32) + 1.0
    p1 = jax.lax.dot_general(
        ones_all, sel, (((1,), (0,)), ((), ())),
        preferred_element_type=jnp.float32)

    deltal = jax.lax.dot_general(
        dmat, dsel_ref[...], (((0,), (0,)), ((), ())),
        preferred_element_type=jnp.float32)
    sq = (x - deltal) ** 2
    t1 = jax.lax.dot_general(
        ones_m, sq, (((1,), (0,)), ((), ())),
        preferred_element_type=jnp.float32)
    dany = jnp.sum(dsel_ref[...], axis=0, keepdims=True)

    out_ref[0:1, :] += t1 * dany
    out_ref[1:2, 0:L] += lse_row
    out_ref[2:3, :] += p1
    out_ref[3:4, 0:L] += ones_m


def _build_consts():
    c = jnp.arange(D, dtype=jnp.int32)
    seg = c // INPUT_DIM
    off = c % INPUT_DIM
    t = jnp.arange(K, dtype=jnp.int32)
    ssel = ((seg[None, :] == t[:, None]) & (off[None, :] > 0)
            ).astype(jnp.float32)
    rrep = (seg[None, :] == t[:, None]).astype(jnp.float32)
    dsel = ((seg[None, :] == t[:, None]) & (off[None, :] == 0)
            ).astype(jnp.float32)
    return ssel, rrep, dsel


_VB = 16
_TSBUF = L + K


def _sc_body(ts_hbm, lbl_hbm, seq_hbm, dmat_hbm, ltgt_hbm,
             ts_v, lbl_v, seq_v, dm_v, lt_v):
    nc = 2
    wid = lax.axis_index("s") * nc + lax.axis_index("c")
    rpw = B // 32
    iota = lax.iota(jnp.int32, _VB)
    zf = jnp.zeros((_VB,), jnp.float32)
    zi = jnp.zeros((_VB,), jnp.int32)
    pltpu.sync_copy(seq_hbm, seq_v.at[pl.ds(0, B)])
    for j in range(rpw):
        row = wid * rpw + j
        ts_v[pl.ds(L - _VB + K, _VB)] = zf
        lbl_v[pl.ds(L - _VB + K, _VB)] = zi
        pltpu.sync_copy(ts_hbm.at[row], ts_v.at[pl.ds(0, L)])
        pltpu.sync_copy(lbl_hbm.at[row], lbl_v.at[pl.ds(0, L)])
        length = seq_v[pl.ds(row, _VB)][0] - K

        def body(k, carry):
            i0 = k * _VB
            valid = (i0 + iota) < length
            for t in range(K):
                a = ts_v[pl.ds(i0 + t, _VB)]
                b2 = ts_v[pl.ds(i0 + t + 1, _VB)]
                lbl = lbl_v[pl.ds(i0 + t + 1, _VB)]
                dm_v[t, pl.ds(i0, _VB)] = jnp.where(valid, b2 - a, 0.0)
                lt = (lbl + (t * INPUT_DIM + 1)).astype(jnp.float32)
                lt_v[t, pl.ds(i0, _VB)] = jnp.where(valid, lt, -1000.0)
            return carry

        lax.fori_loop(0, L // _VB, body, 0)
        pltpu.sync_copy(dm_v, dmat_hbm.at[row])
        pltpu.sync_copy(lt_v, ltgt_hbm.at[row])


@functools.partial(
    pl.kernel,
    mesh=plsc.VectorSubcoreMesh(core_axis_name="c", subcore_axis_name="s"),
    out_type=[
        jax.ShapeDtypeStruct((B, K, L), jnp.float32),
        jax.ShapeDtypeStruct((B, K, L), jnp.float32),
    ],
    scratch_types=[
        pltpu.VMEM((_TSBUF,), jnp.float32),
        pltpu.VMEM((_TSBUF,), jnp.int32),
        pltpu.VMEM((B + _VB,), jnp.int32),
        pltpu.VMEM((K, L), jnp.float32),
        pltpu.VMEM((K, L), jnp.float32),
    ],
)
def _build_aux_sc(ts_hbm, lbl_hbm, seq_hbm, dmat_hbm, ltgt_hbm,
                  ts_v, lbl_v, seq_v, dm_v, lt_v):
    _sc_body(ts_hbm, lbl_hbm, seq_hbm, dmat_hbm, ltgt_hbm,
             ts_v, lbl_v, seq_v, dm_v, lt_v)


def kernel(timestamps, labels, seq_lens, predictions):
    ssel, rrep, dsel = _build_consts()
    dmat, ltgt = _build_aux_sc(timestamps, labels.astype(jnp.int32),
                               seq_lens.astype(jnp.int32))
    sums = pl.pallas_call(
        _tc_body,
        grid=(B // ROWS_PER_STEP,),
        in_specs=[
            pl.BlockSpec(memory_space=pltpu.SMEM),
            pl.BlockSpec((ROWS_PER_STEP, L, D), lambda b: (b, 0, 0)),
            pl.BlockSpec((ROWS_PER_STEP, K, L), lambda b: (b, 0, 0)),
            pl.BlockSpec((ROWS_PER_STEP, K, L), lambda b: (b, 0, 0)),
            pl.BlockSpec((K, D), lambda b: (0, 0)),
            pl.BlockSpec((K, D), lambda b: (0, 0)),
            pl.BlockSpec((K, D), lambda b: (0, 0)),
        ],
        out_specs=pl.BlockSpec((4, D), lambda b: (0, 0)),
        out_shape=jax.ShapeDtypeStruct((4, D), jnp.float32),
    )(seq_lens.astype(jnp.int32), predictions, dmat, ltgt, ssel, rrep, dsel)
    ts_num = jnp.sum(sums[0])
    lbl_num = jnp.sum(sums[1]) - jnp.sum(sums[2])
    n = jnp.sum(sums[3])
    return jnp.stack([ts_num, lbl_num]) / (n * jnp.float32(K))
